# Initial kernel scaffold; baseline (speedup 1.0000x reference)
import jax
import jax.numpy as jnp
from jax.experimental import pallas as pl


def kernel(x_resting, edge_index_resting, pos_resting, x_rigid, edge_index_rigid, Wr0, br0, Wr1, br1, Wg0, bg0, Wg1, bg1, Wm, bm, Wd0, bd0, Wd1, bd1, Wdf, bdf):
    raise NotImplementedError("write your pallas kernel here")


# speedup vs baseline: 10.3237x; 10.3237x over previous
import functools

import jax
import jax.numpy as jnp
from jax import lax
from jax.experimental import pallas as pl
from jax.experimental.pallas import tpu as pltpu
from jax.experimental.pallas import tpu_sc as plsc

NR = 10000
NG = 2048
D = 128
HEADS = 4
ER = 160000
EG = 32768

NC, NS = 2, 16
NW = NC * NS

NRP = 10240
NGA = 2176
ERP = 163840
CH_R = ERP // (NW * 128)
CH_G = EG // (NW * 128)
RB = 512

_MESH = dict(core_axis_name="c", subcore_axis_name="s")


def _sc_degree(dst_r, dst_g, ones128, zr, zg):
    @functools.partial(
        pl.kernel,
        mesh=plsc.VectorSubcoreMesh(**_MESH),
        out_type=(jax.ShapeDtypeStruct((NC, NRP, D), jnp.float32),
                  jax.ShapeDtypeStruct((NC, NGA, D), jnp.float32)),
        scratch_types=[
            pltpu.VMEM((CH_R, 128), jnp.int32),
            pltpu.VMEM((CH_G, 128), jnp.int32),
            pltpu.VMEM((128, D), jnp.float32),
            pltpu.VMEM_SHARED((NRP, D), jnp.float32),
            pltpu.VMEM_SHARED((NGA, D), jnp.float32),
        ],
    )
    def k(dst_r_hbm, dst_g_hbm, ones_hbm, zr_hbm, zg_hbm, outr_hbm, outg_hbm,
          idxr_v, idxg_v, ones_v, accr_sh, accg_sh):
        c = lax.axis_index("c")
        s = lax.axis_index("s")
        wid = c * NS + s
        pltpu.sync_copy(zr_hbm.at[pl.ds(s * (NRP // NS), NRP // NS)],
                        accr_sh.at[pl.ds(s * (NRP // NS), NRP // NS)])
        pltpu.sync_copy(zg_hbm.at[pl.ds(s * (NGA // NS), NGA // NS)],
                        accg_sh.at[pl.ds(s * (NGA // NS), NGA // NS)])
        pltpu.sync_copy(ones_hbm, ones_v)
        pltpu.sync_copy(dst_r_hbm.at[pl.ds(wid * CH_R, CH_R)], idxr_v)
        pltpu.sync_copy(dst_g_hbm.at[pl.ds(wid * CH_G, CH_G)], idxg_v)
        plsc.subcore_barrier()

        @pl.loop(0, CH_R)
        def _(j):
            pltpu.sync_copy(ones_v, accr_sh.at[idxr_v.at[j]], add=True)

        @pl.loop(0, CH_G)
        def _(j):
            pltpu.sync_copy(ones_v, accg_sh.at[idxg_v.at[j]], add=True)

        plsc.subcore_barrier()
        pltpu.sync_copy(accr_sh.at[pl.ds(s * (NRP // NS), NRP // NS)],
                        outr_hbm.at[c, pl.ds(s * (NRP // NS), NRP // NS)])
        pltpu.sync_copy(accg_sh.at[pl.ds(s * (NGA // NS), NGA // NS)],
                        outg_hbm.at[c, pl.ds(s * (NGA // NS), NGA // NS)])

    return k(dst_r, dst_g, ones128, zr, zg)


def _sc_messages(hs_r, src_r, dst_r, hs_g, src_g, dst_g, zr, zg):
    @functools.partial(
        pl.kernel,
        mesh=plsc.VectorSubcoreMesh(**_MESH),
        out_type=(jax.ShapeDtypeStruct((NC, NRP, D), jnp.float32),
                  jax.ShapeDtypeStruct((NC, NGA, D), jnp.float32)),
        scratch_types=[
            pltpu.VMEM((CH_R, 128), jnp.int32),
            pltpu.VMEM((CH_R, 128), jnp.int32),
            pltpu.VMEM((CH_G, 128), jnp.int32),
            pltpu.VMEM((CH_G, 128), jnp.int32),
            pltpu.VMEM((128, D), jnp.float32),
            pltpu.VMEM_SHARED((NRP, D), jnp.float32),
            pltpu.VMEM_SHARED((NGA, D), jnp.float32),
        ],
    )
    def k(hsr_hbm, srcr_hbm, dstr_hbm, hsg_hbm, srcg_hbm, dstg_hbm,
          zr_hbm, zg_hbm, outr_hbm, outg_hbm,
          srcr_v, dstr_v, srcg_v, dstg_v, rows_v, accr_sh, accg_sh):
        c = lax.axis_index("c")
        s = lax.axis_index("s")
        wid = c * NS + s
        pltpu.sync_copy(zr_hbm.at[pl.ds(s * (NRP // NS), NRP // NS)],
                        accr_sh.at[pl.ds(s * (NRP // NS), NRP // NS)])
        pltpu.sync_copy(zg_hbm.at[pl.ds(s * (NGA // NS), NGA // NS)],
                        accg_sh.at[pl.ds(s * (NGA // NS), NGA // NS)])
        pltpu.sync_copy(srcr_hbm.at[pl.ds(wid * CH_R, CH_R)], srcr_v)
        pltpu.sync_copy(dstr_hbm.at[pl.ds(wid * CH_R, CH_R)], dstr_v)
        pltpu.sync_copy(srcg_hbm.at[pl.ds(wid * CH_G, CH_G)], srcg_v)
        pltpu.sync_copy(dstg_hbm.at[pl.ds(wid * CH_G, CH_G)], dstg_v)
        plsc.subcore_barrier()

        @pl.loop(0, CH_R)
        def _(j):
            pltpu.sync_copy(hsr_hbm.at[srcr_v.at[j]], rows_v)
            pltpu.sync_copy(rows_v, accr_sh.at[dstr_v.at[j]], add=True)

        @pl.loop(0, CH_G)
        def _(j):
            pltpu.sync_copy(hsg_hbm.at[srcg_v.at[j]], rows_v)
            pltpu.sync_copy(rows_v, accg_sh.at[dstg_v.at[j]], add=True)

        plsc.subcore_barrier()
        pltpu.sync_copy(accr_sh.at[pl.ds(s * (NRP // NS), NRP // NS)],
                        outr_hbm.at[c, pl.ds(s * (NRP // NS), NRP // NS)])
        pltpu.sync_copy(accg_sh.at[pl.ds(s * (NGA // NS), NGA // NS)],
                        outg_hbm.at[c, pl.ds(s * (NGA // NS), NGA // NS)])

    return k(hs_r, src_r, dst_r, hs_g, src_g, dst_g, zr, zg)


def _dis_from(degp_ref):
    deg = degp_ref[0, :, 0:1] + degp_ref[1, :, 0:1] + 1.0
    return lax.rsqrt(deg)


def _tc_pre(x, w, degp):
    n = x.shape[0]

    def body(x_ref, w_ref, degp_ref, hs_ref, h_ref):
        h = jnp.dot(x_ref[...], w_ref[...], preferred_element_type=jnp.float32)
        dis = _dis_from(degp_ref)
        h_ref[...] = h
        hs_ref[...] = h * dis

    return pl.pallas_call(
        body,
        grid=(n // RB,),
        in_specs=[
            pl.BlockSpec((RB, D), lambda i: (i, 0)),
            pl.BlockSpec((D, D), lambda i: (0, 0)),
            pl.BlockSpec((2, RB, D), lambda i: (0, i, 0)),
        ],
        out_specs=[
            pl.BlockSpec((RB, D), lambda i: (i, 0)),
            pl.BlockSpec((RB, D), lambda i: (i, 0)),
        ],
        out_shape=[jax.ShapeDtypeStruct((n, D), jnp.float32),
                   jax.ShapeDtypeStruct((n, D), jnp.float32)],
    )(x, w, degp)


def _tc_mid(accp, h1, degp, b0, w1):
    n = h1.shape[0]

    def body(accp_ref, h1_ref, degp_ref, b0_ref, w1_ref, hs2_ref, h2_ref):
        dis = _dis_from(degp_ref)
        x1 = jnp.maximum(
            dis * (accp_ref[0] + accp_ref[1]) + h1_ref[...] * (dis * dis)
            + b0_ref[...], 0.0)
        h2 = jnp.dot(x1, w1_ref[...], preferred_element_type=jnp.float32)
        h2_ref[...] = h2
        hs2_ref[...] = h2 * dis

    return pl.pallas_call(
        body,
        grid=(n // RB,),
        in_specs=[
            pl.BlockSpec((2, RB, D), lambda i: (0, i, 0)),
            pl.BlockSpec((RB, D), lambda i: (i, 0)),
            pl.BlockSpec((2, RB, D), lambda i: (0, i, 0)),
            pl.BlockSpec((1, D), lambda i: (0, 0)),
            pl.BlockSpec((D, D), lambda i: (0, 0)),
        ],
        out_specs=[
            pl.BlockSpec((RB, D), lambda i: (i, 0)),
            pl.BlockSpec((RB, D), lambda i: (i, 0)),
        ],
        out_shape=[jax.ShapeDtypeStruct((n, D), jnp.float32),
                   jax.ShapeDtypeStruct((n, D), jnp.float32)],
    )(accp, h1, degp, b0, w1)


def _tc_rigid_final(accp, h2, degp, b1, wm, bm):
    def body(accp_ref, h2_ref, degp_ref, b1_ref, wm_ref, bm_ref,
             xg_ref, qg_ref):
        dis = _dis_from(degp_ref)
        xg = jnp.maximum(
            dis * (accp_ref[0] + accp_ref[1]) + h2_ref[...] * (dis * dis)
            + b1_ref[...], 0.0)
        xg_ref[...] = xg
        for i in range(HEADS):
            qg_ref[i] = jnp.dot(xg, wm_ref[i],
                                preferred_element_type=jnp.float32) + bm_ref[i]

    return pl.pallas_call(
        body,
        grid=(1,),
        in_specs=[
            pl.BlockSpec((2, NG, D), lambda i: (0, 0, 0)),
            pl.BlockSpec((NG, D), lambda i: (0, 0)),
            pl.BlockSpec((2, NG, D), lambda i: (0, 0, 0)),
            pl.BlockSpec((1, D), lambda i: (0, 0)),
            pl.BlockSpec((HEADS, D, D), lambda i: (0, 0, 0)),
            pl.BlockSpec((HEADS, 1, D), lambda i: (0, 0, 0)),
        ],
        out_specs=[
            pl.BlockSpec((NG, D), lambda i: (0, 0)),
            pl.BlockSpec((HEADS, NG, D), lambda i: (0, 0, 0)),
        ],
        out_shape=[jax.ShapeDtypeStruct((NG, D), jnp.float32),
                   jax.ShapeDtypeStruct((HEADS, NG, D), jnp.float32)],
    )(accp, h2, degp, b1, wm, bm)


def _tc_attn_decode(accp, h2, degp, b1, wm, bm, qg, xg,
                    wd0, bd0, wd1, bd1, wdf, bdf, pos):
    def body(accp_ref, h2_ref, degp_ref, b1_ref, wm_ref, bm_ref, qg_ref,
             xg_ref, wd0_ref, bd0_ref, wd1_ref, bd1_ref, wdf_ref, bdf_ref,
             pos_ref, out_ref):
        dis = _dis_from(degp_ref)
        xr = jnp.maximum(
            dis * (accp_ref[0] + accp_ref[1]) + h2_ref[...] * (dis * dis)
            + b1_ref[...], 0.0)
        xgv = xg_ref[...]
        acc = jnp.dot(xr, wd0_ref[0:D], preferred_element_type=jnp.float32)
        for i in range(HEADS):
            qr = jnp.dot(xr, wm_ref[i],
                         preferred_element_type=jnp.float32) + bm_ref[i]
            s = lax.dot_general(qr, qg_ref[i], (((1,), (1,)), ((), ())),
                                preferred_element_type=jnp.float32)
            m = jnp.max(s, axis=1, keepdims=True)
            p = jnp.exp(s - m)
            denom = jnp.sum(p, axis=1, keepdims=True)
            o = jnp.dot(p, xgv, preferred_element_type=jnp.float32) / denom
            acc = acc + jnp.dot(o, wd0_ref[D * (i + 1):D * (i + 2)],
                                preferred_element_type=jnp.float32)
        hd = jnp.maximum(acc + bd0_ref[...], 0.0)
        hd = jnp.maximum(
            jnp.dot(hd, wd1_ref[...], preferred_element_type=jnp.float32)
            + bd1_ref[...], 0.0)
        out_ref[...] = (jnp.dot(hd, wdf_ref[...],
                                preferred_element_type=jnp.float32)
                        + bdf_ref[...] + pos_ref[...])

    return pl.pallas_call(
        body,
        grid=(NRP // RB,),
        in_specs=[
            pl.BlockSpec((2, RB, D), lambda i: (0, i, 0)),
            pl.BlockSpec((RB, D), lambda i: (i, 0)),
            pl.BlockSpec((2, RB, D), lambda i: (0, i, 0)),
            pl.BlockSpec((1, D), lambda i: (0, 0)),
            pl.BlockSpec((HEADS, D, D), lambda i: (0, 0, 0)),
            pl.BlockSpec((HEADS, 1, D), lambda i: (0, 0, 0)),
            pl.BlockSpec((HEADS, NG, D), lambda i: (0, 0, 0)),
            pl.BlockSpec((NG, D), lambda i: (0, 0)),
            pl.BlockSpec((D * (HEADS + 1), D), lambda i: (0, 0)),
            pl.BlockSpec((1, D), lambda i: (0, 0)),
            pl.BlockSpec((D, D), lambda i: (0, 0)),
            pl.BlockSpec((1, D), lambda i: (0, 0)),
            pl.BlockSpec((D, 8), lambda i: (0, 0)),
            pl.BlockSpec((1, 8), lambda i: (0, 0)),
            pl.BlockSpec((RB, 8), lambda i: (i, 0)),
        ],
        out_specs=pl.BlockSpec((RB, 8), lambda i: (i, 0)),
        out_shape=jax.ShapeDtypeStruct((NRP, 8), jnp.float32),
    )(accp, h2, degp, b1, wm, bm, qg, xg, wd0, bd0, wd1, bd1, wdf, bdf, pos)


def kernel(x_resting, edge_index_resting, pos_resting, x_rigid,
           edge_index_rigid, Wr0, br0, Wr1, br1, Wg0, bg0, Wg1, bg1, Wm, bm,
           Wd0, bd0, Wd1, bd1, Wdf, bdf):
    f32 = jnp.float32

    src_r, dst_r = edge_index_resting[0], edge_index_resting[1]
    npad = ERP - ER
    pad_ar = jnp.arange(npad, dtype=jnp.int32)
    src_rp = jnp.concatenate([src_r, pad_ar % NR]).reshape(ERP // 128, 128)
    dst_rp = jnp.concatenate([dst_r, NR + pad_ar % (NRP - NR)]
                             ).reshape(ERP // 128, 128)
    src_g = edge_index_rigid[0].reshape(EG // 128, 128)
    dst_g = edge_index_rigid[1].reshape(EG // 128, 128)

    ones128 = jnp.ones((128, D), f32)
    zr = jnp.zeros((NRP, D), f32)
    zg = jnp.zeros((NGA, D), f32)

    degp_r, degp_g = _sc_degree(dst_rp, dst_g, ones128, zr, zg)
    degp_g = degp_g[:, :NG]

    x_r = jnp.pad(x_resting, ((0, NRP - NR), (0, 0)))
    hs1_r, h1_r = _tc_pre(x_r, Wr0, degp_r)
    hs1_g, h1_g = _tc_pre(x_rigid, Wg0, degp_g)
    acc1_r, acc1_g = _sc_messages(hs1_r, src_rp, dst_rp, hs1_g, src_g, dst_g,
                                  zr, zg)
    hs2_r, h2_r = _tc_mid(acc1_r, h1_r, degp_r, br0.reshape(1, D), Wr1)
    hs2_g, h2_g = _tc_mid(acc1_g[:, :NG], h1_g, degp_g, bg0.reshape(1, D),
                          Wg1)
    acc2_r, acc2_g = _sc_messages(hs2_r, src_rp, dst_rp, hs2_g, src_g, dst_g,
                                  zr, zg)

    xg, qg = _tc_rigid_final(acc2_g[:, :NG], h2_g, degp_g,
                             bg1.reshape(1, D), Wm, bm.reshape(HEADS, 1, D))

    wdf_p = jnp.pad(Wdf, ((0, 0), (0, 8 - Wdf.shape[1])))
    bdf_p = jnp.pad(bdf, (0, 8 - bdf.shape[0])).reshape(1, 8)
    pos_p = jnp.pad(pos_resting, ((0, NRP - NR), (0, 8 - pos_resting.shape[1])))
    out = _tc_attn_decode(acc2_r, h2_r, degp_r, br1.reshape(1, D), Wm,
                          bm.reshape(HEADS, 1, D), qg, xg, Wd0,
                          bd0.reshape(1, D), Wd1, bd1.reshape(1, D),
                          wdf_p, bdf_p, pos_p)
    return out[:NR, :3]


# speedup vs baseline: 11.3140x; 1.0959x over previous
import functools

import jax
import jax.numpy as jnp
from jax import lax
from jax.experimental import pallas as pl
from jax.experimental.pallas import tpu as pltpu
from jax.experimental.pallas import tpu_sc as plsc

NR = 10000
NG = 2048
D = 128
HEADS = 4
ER = 160000
EG = 32768

NC, NS = 2, 16
NW = NC * NS

NRP = 10240
NGA = 2176
ERP = 163840
CH_R = ERP // (NW * 128)
CH_G = EG // (NW * 128)
RB = 512

_MESH = dict(core_axis_name="c", subcore_axis_name="s")


def _sc_degree(dst_r, dst_g, ones128, zr, zg):
    @functools.partial(
        pl.kernel,
        mesh=plsc.VectorSubcoreMesh(**_MESH),
        out_type=(jax.ShapeDtypeStruct((NC, NRP, D), jnp.float32),
                  jax.ShapeDtypeStruct((NC, NGA, D), jnp.float32)),
        scratch_types=[
            pltpu.VMEM((CH_R, 128), jnp.int32),
            pltpu.VMEM((CH_G, 128), jnp.int32),
            pltpu.VMEM((128, D), jnp.float32),
            pltpu.VMEM_SHARED((NRP, D), jnp.float32),
            pltpu.VMEM_SHARED((NGA, D), jnp.float32),
            pltpu.SemaphoreType.DMA,
        ],
    )
    def k(dst_r_hbm, dst_g_hbm, ones_hbm, zr_hbm, zg_hbm, outr_hbm, outg_hbm,
          idxr_v, idxg_v, ones_v, accr_sh, accg_sh, sem):
        c = lax.axis_index("c")
        s = lax.axis_index("s")
        wid = c * NS + s
        pltpu.sync_copy(zr_hbm.at[pl.ds(s * (NRP // NS), NRP // NS)],
                        accr_sh.at[pl.ds(s * (NRP // NS), NRP // NS)])
        pltpu.sync_copy(zg_hbm.at[pl.ds(s * (NGA // NS), NGA // NS)],
                        accg_sh.at[pl.ds(s * (NGA // NS), NGA // NS)])
        pltpu.sync_copy(ones_hbm, ones_v)
        pltpu.sync_copy(dst_r_hbm.at[pl.ds(wid * CH_R, CH_R)], idxr_v)
        pltpu.sync_copy(dst_g_hbm.at[pl.ds(wid * CH_G, CH_G)], idxg_v)
        plsc.subcore_barrier()

        @pl.loop(0, CH_R // 4)
        def _(i):
            j = 4 * i
            ds_ = [pltpu.async_copy(ones_v, accr_sh.at[idxr_v.at[j + t]],
                                    sem, add=True) for t in range(4)]
            for d in ds_:
                d.wait()

        @pl.loop(0, CH_G // 4)
        def _(i):
            j = 4 * i
            ds_ = [pltpu.async_copy(ones_v, accg_sh.at[idxg_v.at[j + t]],
                                    sem, add=True) for t in range(4)]
            for d in ds_:
                d.wait()

        plsc.subcore_barrier()
        pltpu.sync_copy(accr_sh.at[pl.ds(s * (NRP // NS), NRP // NS)],
                        outr_hbm.at[c, pl.ds(s * (NRP // NS), NRP // NS)])
        pltpu.sync_copy(accg_sh.at[pl.ds(s * (NGA // NS), NGA // NS)],
                        outg_hbm.at[c, pl.ds(s * (NGA // NS), NGA // NS)])

    return k(dst_r, dst_g, ones128, zr, zg)


def _sc_messages(hs, src, dst, zeros, nacc, nch):
    rpt = nacc // NS

    @functools.partial(
        pl.kernel,
        mesh=plsc.VectorSubcoreMesh(**_MESH),
        out_type=jax.ShapeDtypeStruct((NC, nacc, D), jnp.float32),
        scratch_types=[
            pltpu.VMEM((nch, 64), jnp.int32),
            pltpu.VMEM((nch, 64), jnp.int32),
            pltpu.VMEM((64, D), jnp.float32),
            pltpu.VMEM((64, D), jnp.float32),
            pltpu.VMEM_SHARED((nacc, D), jnp.float32),
            pltpu.SemaphoreType.DMA,
        ],
    )
    def k(hs_hbm, src_hbm, dst_hbm, z_hbm, out_hbm,
          src_v, dst_v, r0, r1, acc_sh, sem):
        c = lax.axis_index("c")
        s = lax.axis_index("s")
        wid = c * NS + s
        pltpu.sync_copy(z_hbm.at[pl.ds(s * rpt, rpt)],
                        acc_sh.at[pl.ds(s * rpt, rpt)])
        pltpu.sync_copy(src_hbm.at[pl.ds(wid * nch, nch)], src_v)
        pltpu.sync_copy(dst_hbm.at[pl.ds(wid * nch, nch)], dst_v)
        plsc.subcore_barrier()

        rows = [r0, r1]

        @pl.loop(0, nch // 2)
        def _(i):
            j = 2 * i
            ds_ = [pltpu.async_copy(hs_hbm.at[src_v.at[j + t]], rows[t], sem)
                   for t in range(2)]
            for t in range(2):
                ds_[t].wait()
                pltpu.sync_copy(rows[t], acc_sh.at[dst_v.at[j + t]], add=True)

        plsc.subcore_barrier()
        pltpu.sync_copy(acc_sh.at[pl.ds(s * rpt, rpt)],
                        out_hbm.at[c, pl.ds(s * rpt, rpt)])

    return k(hs, src, dst, zeros)


def _dis_from(degp_ref):
    deg = degp_ref[0, :, 0:1] + degp_ref[1, :, 0:1] + 1.0
    return lax.rsqrt(deg)


def _tc_pre(x, w, degp):
    n = x.shape[0]

    def body(x_ref, w_ref, degp_ref, hs_ref, h_ref):
        h = jnp.dot(x_ref[...], w_ref[...], preferred_element_type=jnp.float32)
        dis = _dis_from(degp_ref)
        h_ref[...] = h
        hs_ref[...] = h * dis

    return pl.pallas_call(
        body,
        grid=(n // RB,),
        in_specs=[
            pl.BlockSpec((RB, D), lambda i: (i, 0)),
            pl.BlockSpec((D, D), lambda i: (0, 0)),
            pl.BlockSpec((2, RB, D), lambda i: (0, i, 0)),
        ],
        out_specs=[
            pl.BlockSpec((RB, D), lambda i: (i, 0)),
            pl.BlockSpec((RB, D), lambda i: (i, 0)),
        ],
        out_shape=[jax.ShapeDtypeStruct((n, D), jnp.float32),
                   jax.ShapeDtypeStruct((n, D), jnp.float32)],
    )(x, w, degp)


def _tc_mid(accp, h1, degp, b0, w1):
    n = h1.shape[0]

    def body(accp_ref, h1_ref, degp_ref, b0_ref, w1_ref, hs2_ref, h2_ref):
        dis = _dis_from(degp_ref)
        x1 = jnp.maximum(
            dis * (accp_ref[0] + accp_ref[1]) + h1_ref[...] * (dis * dis)
            + b0_ref[...], 0.0)
        h2 = jnp.dot(x1, w1_ref[...], preferred_element_type=jnp.float32)
        h2_ref[...] = h2
        hs2_ref[...] = h2 * dis

    return pl.pallas_call(
        body,
        grid=(n // RB,),
        in_specs=[
            pl.BlockSpec((2, RB, D), lambda i: (0, i, 0)),
            pl.BlockSpec((RB, D), lambda i: (i, 0)),
            pl.BlockSpec((2, RB, D), lambda i: (0, i, 0)),
            pl.BlockSpec((1, D), lambda i: (0, 0)),
            pl.BlockSpec((D, D), lambda i: (0, 0)),
        ],
        out_specs=[
            pl.BlockSpec((RB, D), lambda i: (i, 0)),
            pl.BlockSpec((RB, D), lambda i: (i, 0)),
        ],
        out_shape=[jax.ShapeDtypeStruct((n, D), jnp.float32),
                   jax.ShapeDtypeStruct((n, D), jnp.float32)],
    )(accp, h1, degp, b0, w1)


def _tc_rigid_final(accp, h2, degp, b1, wm, bm):
    def body(accp_ref, h2_ref, degp_ref, b1_ref, wm_ref, bm_ref,
             xg_ref, qg_ref):
        dis = _dis_from(degp_ref)
        xg = jnp.maximum(
            dis * (accp_ref[0] + accp_ref[1]) + h2_ref[...] * (dis * dis)
            + b1_ref[...], 0.0)
        xg_ref[...] = xg
        for i in range(HEADS):
            qg_ref[i] = jnp.dot(xg, wm_ref[i],
                                preferred_element_type=jnp.float32) + bm_ref[i]

    return pl.pallas_call(
        body,
        grid=(1,),
        in_specs=[
            pl.BlockSpec((2, NG, D), lambda i: (0, 0, 0)),
            pl.BlockSpec((NG, D), lambda i: (0, 0)),
            pl.BlockSpec((2, NG, D), lambda i: (0, 0, 0)),
            pl.BlockSpec((1, D), lambda i: (0, 0)),
            pl.BlockSpec((HEADS, D, D), lambda i: (0, 0, 0)),
            pl.BlockSpec((HEADS, 1, D), lambda i: (0, 0, 0)),
        ],
        out_specs=[
            pl.BlockSpec((NG, D), lambda i: (0, 0)),
            pl.BlockSpec((HEADS, NG, D), lambda i: (0, 0, 0)),
        ],
        out_shape=[jax.ShapeDtypeStruct((NG, D), jnp.float32),
                   jax.ShapeDtypeStruct((HEADS, NG, D), jnp.float32)],
    )(accp, h2, degp, b1, wm, bm)


def _tc_attn_decode(accp, h2, degp, b1, wm, bm, qg, xg,
                    wd0, bd0, wd1, bd1, wdf, bdf, pos):
    def body(accp_ref, h2_ref, degp_ref, b1_ref, wm_ref, bm_ref, qg_ref,
             xg_ref, wd0_ref, bd0_ref, wd1_ref, bd1_ref, wdf_ref, bdf_ref,
             pos_ref, out_ref):
        dis = _dis_from(degp_ref)
        xr = jnp.maximum(
            dis * (accp_ref[0] + accp_ref[1]) + h2_ref[...] * (dis * dis)
            + b1_ref[...], 0.0)
        xgv = xg_ref[...]
        acc = jnp.dot(xr, wd0_ref[0:D], preferred_element_type=jnp.float32)
        for i in range(HEADS):
            qr = jnp.dot(xr, wm_ref[i],
                         preferred_element_type=jnp.float32) + bm_ref[i]
            s = lax.dot_general(qr, qg_ref[i], (((1,), (1,)), ((), ())),
                                preferred_element_type=jnp.float32)
            m = jnp.max(s, axis=1, keepdims=True)
            p = jnp.exp(s - m)
            denom = jnp.sum(p, axis=1, keepdims=True)
            o = jnp.dot(p, xgv, preferred_element_type=jnp.float32) / denom
            acc = acc + jnp.dot(o, wd0_ref[D * (i + 1):D * (i + 2)],
                                preferred_element_type=jnp.float32)
        hd = jnp.maximum(acc + bd0_ref[...], 0.0)
        hd = jnp.maximum(
            jnp.dot(hd, wd1_ref[...], preferred_element_type=jnp.float32)
            + bd1_ref[...], 0.0)
        out_ref[...] = (jnp.dot(hd, wdf_ref[...],
                                preferred_element_type=jnp.float32)
                        + bdf_ref[...] + pos_ref[...])

    return pl.pallas_call(
        body,
        grid=(NRP // RB,),
        in_specs=[
            pl.BlockSpec((2, RB, D), lambda i: (0, i, 0)),
            pl.BlockSpec((RB, D), lambda i: (i, 0)),
            pl.BlockSpec((2, RB, D), lambda i: (0, i, 0)),
            pl.BlockSpec((1, D), lambda i: (0, 0)),
            pl.BlockSpec((HEADS, D, D), lambda i: (0, 0, 0)),
            pl.BlockSpec((HEADS, 1, D), lambda i: (0, 0, 0)),
            pl.BlockSpec((HEADS, NG, D), lambda i: (0, 0, 0)),
            pl.BlockSpec((NG, D), lambda i: (0, 0)),
            pl.BlockSpec((D * (HEADS + 1), D), lambda i: (0, 0)),
            pl.BlockSpec((1, D), lambda i: (0, 0)),
            pl.BlockSpec((D, D), lambda i: (0, 0)),
            pl.BlockSpec((1, D), lambda i: (0, 0)),
            pl.BlockSpec((D, 8), lambda i: (0, 0)),
            pl.BlockSpec((1, 8), lambda i: (0, 0)),
            pl.BlockSpec((RB, 8), lambda i: (i, 0)),
        ],
        out_specs=pl.BlockSpec((RB, 8), lambda i: (i, 0)),
        out_shape=jax.ShapeDtypeStruct((NRP, 8), jnp.float32),
    )(accp, h2, degp, b1, wm, bm, qg, xg, wd0, bd0, wd1, bd1, wdf, bdf, pos)


def kernel(x_resting, edge_index_resting, pos_resting, x_rigid,
           edge_index_rigid, Wr0, br0, Wr1, br1, Wg0, bg0, Wg1, bg1, Wm, bm,
           Wd0, bd0, Wd1, bd1, Wdf, bdf):
    f32 = jnp.float32

    src_r, dst_r = edge_index_resting[0], edge_index_resting[1]
    npad = ERP - ER
    pad_ar = jnp.arange(npad, dtype=jnp.int32)
    src_rf = jnp.concatenate([src_r, pad_ar % NR])
    dst_rf = jnp.concatenate([dst_r, NR + pad_ar % (NRP - NR)])
    src_rp = src_rf.reshape(ERP // 128, 128)
    dst_rp = dst_rf.reshape(ERP // 128, 128)
    src_r64 = src_rf.reshape(ERP // 64, 64)
    dst_r64 = dst_rf.reshape(ERP // 64, 64)
    src_g64 = edge_index_rigid[0].reshape(EG // 64, 64)
    dst_g64 = edge_index_rigid[1].reshape(EG // 64, 64)
    dst_g = edge_index_rigid[1].reshape(EG // 128, 128)

    ones128 = jnp.ones((128, D), f32)
    zr = jnp.zeros((NRP, D), f32)
    zg = jnp.zeros((NGA, D), f32)

    degp_r, degp_g = _sc_degree(dst_rp, dst_g, ones128, zr, zg)
    degp_g = degp_g[:, :NG]

    x_r = jnp.pad(x_resting, ((0, NRP - NR), (0, 0)))
    hs1_r, h1_r = _tc_pre(x_r, Wr0, degp_r)
    hs1_g, h1_g = _tc_pre(x_rigid, Wg0, degp_g)
    acc1_g = _sc_messages(hs1_g, src_g64, dst_g64, zg, NGA, EG // (NW * 64))
    acc1_r = _sc_messages(hs1_r, src_r64, dst_r64, zr, NRP, ERP // (NW * 64))
    hs2_r, h2_r = _tc_mid(acc1_r, h1_r, degp_r, br0.reshape(1, D), Wr1)
    hs2_g, h2_g = _tc_mid(acc1_g[:, :NG], h1_g, degp_g, bg0.reshape(1, D),
                          Wg1)
    acc2_g = _sc_messages(hs2_g, src_g64, dst_g64, zg, NGA, EG // (NW * 64))
    acc2_r = _sc_messages(hs2_r, src_r64, dst_r64, zr, NRP, ERP // (NW * 64))

    xg, qg = _tc_rigid_final(acc2_g[:, :NG], h2_g, degp_g,
                             bg1.reshape(1, D), Wm, bm.reshape(HEADS, 1, D))

    wdf_p = jnp.pad(Wdf, ((0, 0), (0, 8 - Wdf.shape[1])))
    bdf_p = jnp.pad(bdf, (0, 8 - bdf.shape[0])).reshape(1, 8)
    pos_p = jnp.pad(pos_resting, ((0, NRP - NR), (0, 8 - pos_resting.shape[1])))
    out = _tc_attn_decode(acc2_r, h2_r, degp_r, br1.reshape(1, D), Wm,
                          bm.reshape(HEADS, 1, D), qg, xg, Wd0,
                          bd0.reshape(1, D), Wd1, bd1.reshape(1, D),
                          wdf_p, bdf_p, pos_p)
    return out[:NR, :3]
